# baseline (speedup 1.0000x reference)
import numpy as np
import jax
import jax.numpy as jnp
from jax import lax
from jax.experimental import pallas as pl
from jax.experimental.pallas import tpu as pltpu

_N, _H, _A = 16384, 64, 8
_K = 512
_GAMMA = 0.99
_DISC = np.power(np.float32(_GAMMA), np.arange(_H, dtype=np.float32))


def _adv_kernel(r_ref, d_ref, o_ref):
    s = jnp.sum(r_ref[...] * d_ref[...], axis=1, keepdims=True)
    o_ref[...] = s.reshape(1, 128, 1)


def _thresh_kernel(a_ref, o_ref):
    ab = lax.bitcast_convert_type(a_ref[...], jnp.int32)

    def body(_, carry):
        lo, hi = carry
        mid = lo + (hi - lo) // 2
        cnt = jnp.sum((ab >= mid).astype(jnp.int32))
        ok = cnt >= _K
        return (jnp.where(ok, mid, lo), jnp.where(ok, hi, mid))

    lo, _hi = lax.fori_loop(
        0, 31, body, (jnp.int32(0), jnp.int32(0x43000000)))
    t = lax.bitcast_convert_type(lo, jnp.float32)
    o_ref[...] = jnp.full((1, 128), t, jnp.float32)


def _wsum_kernel(t_ref, a_ref, act_ref, m_ref, s_ref, s0_ref, s1_ref, s2_ref):
    i = pl.program_id(0)
    t = t_ref[0]
    a = a_ref[...].reshape(1, 2048)
    w = jnp.exp(a) * (a >= t).astype(jnp.float32)
    blk = act_ref[...]
    p1 = jnp.dot(w, blk, preferred_element_type=jnp.float32)
    p2 = jnp.dot(w, blk * blk, preferred_element_type=jnp.float32)
    p0 = jnp.sum(w)

    @pl.when(i == 0)
    def _init():
        s0_ref[...] = jnp.zeros_like(s0_ref)
        s1_ref[...] = jnp.zeros_like(s1_ref)
        s2_ref[...] = jnp.zeros_like(s2_ref)

    s0_ref[...] += jnp.full(s0_ref.shape, p0, jnp.float32)
    s1_ref[...] += p1
    s2_ref[...] += p2

    @pl.when(i == pl.num_programs(0) - 1)
    def _fini():
        s0 = s0_ref[0, 0]
        mean = s1_ref[...] / s0
        m_ref[...] = mean
        s_ref[...] = jnp.sqrt(jnp.maximum(s2_ref[...] / s0 - mean * mean, 0.0))


def kernel(actions, rewards):
    r2 = rewards.reshape(_N, _H)
    a2 = actions.reshape(_N, _H * _A)
    disc = jnp.asarray(_DISC).reshape(1, _H)

    adv3 = pl.pallas_call(
        _adv_kernel,
        grid=(128,),
        in_specs=[
            pl.BlockSpec((128, _H), lambda i: (i, 0)),
            pl.BlockSpec((1, _H), lambda i: (0, 0)),
        ],
        out_specs=pl.BlockSpec((1, 128, 1), lambda i: (i, 0, 0)),
        out_shape=jax.ShapeDtypeStruct((128, 128, 1), jnp.float32),
    )(r2, disc)

    thresh = pl.pallas_call(
        _thresh_kernel,
        in_specs=[pl.BlockSpec((128, 128), lambda: (0, 0))],
        out_specs=pl.BlockSpec((1, 128), lambda: (0, 0)),
        out_shape=jax.ShapeDtypeStruct((1, 128), jnp.float32),
    )(adv3.reshape(128, 128))

    adv_row = adv3.reshape(8, 1, 2048)
    means, stds = pl.pallas_call(
        _wsum_kernel,
        grid=(8,),
        in_specs=[
            pl.BlockSpec(memory_space=pltpu.SMEM),
            pl.BlockSpec((1, 1, 2048), lambda i: (i, 0, 0)),
            pl.BlockSpec((2048, _H * _A), lambda i: (i, 0)),
        ],
        out_specs=[
            pl.BlockSpec((1, _H * _A), lambda i: (0, 0)),
            pl.BlockSpec((1, _H * _A), lambda i: (0, 0)),
        ],
        out_shape=[
            jax.ShapeDtypeStruct((1, _H * _A), jnp.float32),
            jax.ShapeDtypeStruct((1, _H * _A), jnp.float32),
        ],
        scratch_shapes=[
            pltpu.VMEM((1, 128), jnp.float32),
            pltpu.VMEM((1, _H * _A), jnp.float32),
            pltpu.VMEM((1, _H * _A), jnp.float32),
        ],
    )(thresh.reshape(128), adv_row, a2)

    return means.reshape(1, _H, _A), stds.reshape(1, _H, _A)


# speedup vs baseline: 1.6928x; 1.6928x over previous
import numpy as np
import jax
import jax.numpy as jnp
from jax import lax
from jax.experimental import pallas as pl
from jax.experimental.pallas import tpu as pltpu

_N, _H, _A = 16384, 64, 8
_K = 512
_GAMMA = 0.99
_DISC = np.power(np.float32(_GAMMA), np.arange(_H, dtype=np.float32))


def _adv_kernel(r_ref, d_ref, o_ref):
    s = jnp.dot(r_ref[...], d_ref[..., 0:1], preferred_element_type=jnp.float32)
    o_ref[...] = s.reshape(1, 2048, 1)


def _thresh_kernel(a_ref, o_ref):
    ab = lax.bitcast_convert_type(a_ref[...], jnp.int32)

    def body(_, carry):
        lo, hi = carry
        mid = lo + (hi - lo) // 2
        cnt = jnp.sum((ab >= mid).astype(jnp.int32))
        ok = cnt >= _K
        return (jnp.where(ok, mid, lo), jnp.where(ok, hi, mid))

    lo, _hi = lax.fori_loop(
        0, 31, body, (jnp.int32(0), jnp.int32(0x43000000)))
    t = lax.bitcast_convert_type(lo, jnp.float32)
    o_ref[...] = jnp.full((1, 128), t, jnp.float32)


def _wsum_kernel(t_ref, a_ref, act_ref, m_ref, s_ref, s0_ref, s1_ref, s2_ref):
    i = pl.program_id(0)
    t = t_ref[0]
    a = a_ref[...].reshape(1, 2048)
    w = jnp.exp(a) * (a >= t).astype(jnp.float32)
    blk = act_ref[...]
    p1 = jnp.dot(w, blk, preferred_element_type=jnp.float32)
    p2 = jnp.dot(w, blk * blk, preferred_element_type=jnp.float32)
    p0 = jnp.sum(w)

    @pl.when(i == 0)
    def _init():
        s0_ref[...] = jnp.zeros_like(s0_ref)
        s1_ref[...] = jnp.zeros_like(s1_ref)
        s2_ref[...] = jnp.zeros_like(s2_ref)

    s0_ref[...] += jnp.full(s0_ref.shape, p0, jnp.float32)
    s1_ref[...] += p1
    s2_ref[...] += p2

    @pl.when(i == pl.num_programs(0) - 1)
    def _fini():
        s0 = s0_ref[0, 0]
        mean = s1_ref[...] / s0
        m_ref[...] = mean
        s_ref[...] = jnp.sqrt(jnp.maximum(s2_ref[...] / s0 - mean * mean, 0.0))


def kernel(actions, rewards):
    r2 = rewards.reshape(_N, _H)
    a2 = actions.reshape(_N, _H * _A)
    disc = jnp.broadcast_to(jnp.asarray(_DISC).reshape(_H, 1), (_H, 128))

    adv3 = pl.pallas_call(
        _adv_kernel,
        grid=(8,),
        in_specs=[
            pl.BlockSpec((2048, _H), lambda i: (i, 0)),
            pl.BlockSpec((_H, 128), lambda i: (0, 0)),
        ],
        out_specs=pl.BlockSpec((1, 2048, 1), lambda i: (i, 0, 0)),
        out_shape=jax.ShapeDtypeStruct((8, 2048, 1), jnp.float32),
    )(r2, disc)

    thresh = pl.pallas_call(
        _thresh_kernel,
        in_specs=[pl.BlockSpec((128, 128), lambda: (0, 0))],
        out_specs=pl.BlockSpec((1, 128), lambda: (0, 0)),
        out_shape=jax.ShapeDtypeStruct((1, 128), jnp.float32),
    )(adv3.reshape(128, 128))

    adv_row = adv3.reshape(8, 1, 2048)
    means, stds = pl.pallas_call(
        _wsum_kernel,
        grid=(8,),
        in_specs=[
            pl.BlockSpec(memory_space=pltpu.SMEM),
            pl.BlockSpec((1, 1, 2048), lambda i: (i, 0, 0)),
            pl.BlockSpec((2048, _H * _A), lambda i: (i, 0)),
        ],
        out_specs=[
            pl.BlockSpec((1, _H * _A), lambda i: (0, 0)),
            pl.BlockSpec((1, _H * _A), lambda i: (0, 0)),
        ],
        out_shape=[
            jax.ShapeDtypeStruct((1, _H * _A), jnp.float32),
            jax.ShapeDtypeStruct((1, _H * _A), jnp.float32),
        ],
        scratch_shapes=[
            pltpu.VMEM((1, 128), jnp.float32),
            pltpu.VMEM((1, _H * _A), jnp.float32),
            pltpu.VMEM((1, _H * _A), jnp.float32),
        ],
    )(thresh.reshape(128), adv_row, a2)

    return means.reshape(1, _H, _A), stds.reshape(1, _H, _A)


# speedup vs baseline: 1.6944x; 1.0009x over previous
import numpy as np
import jax
import jax.numpy as jnp
from jax import lax
from jax.experimental import pallas as pl
from jax.experimental.pallas import tpu as pltpu

_N, _H, _A = 16384, 64, 8
_K = 512
_GAMMA = 0.99


def _adv_kernel(r_ref, d_ref, o_ref):
    s = jnp.sum(r_ref[...] * d_ref[...], axis=1, keepdims=True)
    o_ref[...] = s.reshape(1, 2048, 1)


_W = 16


def _thresh_kernel(a_ref, o_ref):
    ab = lax.bitcast_convert_type(a_ref[...], jnp.int32)

    def body(_, carry):
        lo, hi = carry
        mid = lo + (hi - lo) // 2
        cnt = jnp.sum((ab >= mid).astype(jnp.int32))
        ok = cnt >= _K
        return (jnp.where(ok, mid, lo), jnp.where(ok, hi, mid))

    lo, _hi = lax.fori_loop(
        0, 31, body, (jnp.int32(0), jnp.int32(0x43000000)))
    t_lo = jnp.maximum(lo - _W, 0)
    t_hi = lo + _W
    n_above = jnp.sum((ab > t_hi).astype(jnp.int32))
    n_amb = jnp.sum(((ab >= t_lo) & (ab <= t_hi)).astype(jnp.int32))
    alpha = (_K - n_above).astype(jnp.float32) / n_amb.astype(jnp.float32)
    lane = lax.broadcasted_iota(jnp.int32, (1, 128), 1)
    tlo_f = lax.bitcast_convert_type(t_lo, jnp.float32)
    thi_f = lax.bitcast_convert_type(t_hi, jnp.float32)
    o_ref[...] = jnp.where(lane == 0, tlo_f,
                           jnp.where(lane == 1, thi_f, alpha))


def _wsum_kernel(t_ref, a_ref, act_ref, m_ref, s_ref, s0_ref, s1_ref, s2_ref):
    i = pl.program_id(0)
    tlo, thi, alpha = t_ref[0], t_ref[1], t_ref[2]
    a = a_ref[...].reshape(1, 2048)
    sel = jnp.where(a > thi, 1.0, jnp.where(a >= tlo, alpha, 0.0))
    w = jnp.exp(a) * sel
    blk = act_ref[...]
    p1 = jnp.dot(w, blk, preferred_element_type=jnp.float32)
    p2 = jnp.dot(w, blk * blk, preferred_element_type=jnp.float32)
    p0 = jnp.sum(w)

    @pl.when(i == 0)
    def _init():
        s0_ref[...] = jnp.zeros_like(s0_ref)
        s1_ref[...] = jnp.zeros_like(s1_ref)
        s2_ref[...] = jnp.zeros_like(s2_ref)

    s0_ref[...] += jnp.full(s0_ref.shape, p0, jnp.float32)
    s1_ref[...] += p1
    s2_ref[...] += p2

    @pl.when(i == pl.num_programs(0) - 1)
    def _fini():
        s0 = s0_ref[0, 0]
        mean = s1_ref[...] / s0
        m_ref[...] = mean
        s_ref[...] = jnp.sqrt(jnp.maximum(s2_ref[...] / s0 - mean * mean, 0.0))


def kernel(actions, rewards):
    r2 = rewards.reshape(_N, _H)
    a2 = actions.reshape(_N, _H * _A)
    disc = (jnp.float32(_GAMMA) **
            jnp.arange(_H, dtype=jnp.float32)).reshape(1, _H)

    adv3 = pl.pallas_call(
        _adv_kernel,
        grid=(8,),
        in_specs=[
            pl.BlockSpec((2048, _H), lambda i: (i, 0)),
            pl.BlockSpec((1, _H), lambda i: (0, 0)),
        ],
        out_specs=pl.BlockSpec((1, 2048, 1), lambda i: (i, 0, 0)),
        out_shape=jax.ShapeDtypeStruct((8, 2048, 1), jnp.float32),
    )(r2, disc)

    thresh = pl.pallas_call(
        _thresh_kernel,
        in_specs=[pl.BlockSpec((128, 128), lambda: (0, 0))],
        out_specs=pl.BlockSpec((1, 128), lambda: (0, 0)),
        out_shape=jax.ShapeDtypeStruct((1, 128), jnp.float32),
    )(adv3.reshape(128, 128))

    adv_row = adv3.reshape(8, 1, 2048)
    means, stds = pl.pallas_call(
        _wsum_kernel,
        grid=(8,),
        in_specs=[
            pl.BlockSpec(memory_space=pltpu.SMEM),
            pl.BlockSpec((1, 1, 2048), lambda i: (i, 0, 0)),
            pl.BlockSpec((2048, _H * _A), lambda i: (i, 0)),
        ],
        out_specs=[
            pl.BlockSpec((1, _H * _A), lambda i: (0, 0)),
            pl.BlockSpec((1, _H * _A), lambda i: (0, 0)),
        ],
        out_shape=[
            jax.ShapeDtypeStruct((1, _H * _A), jnp.float32),
            jax.ShapeDtypeStruct((1, _H * _A), jnp.float32),
        ],
        scratch_shapes=[
            pltpu.VMEM((1, 128), jnp.float32),
            pltpu.VMEM((1, _H * _A), jnp.float32),
            pltpu.VMEM((1, _H * _A), jnp.float32),
        ],
    )(thresh.reshape(128), adv_row, a2)

    return means.reshape(1, _H, _A), stds.reshape(1, _H, _A)
